# Initial kernel scaffold; baseline (speedup 1.0000x reference)
import jax
import jax.numpy as jnp
from jax.experimental import pallas as pl


def kernel(x, table, W, b):
    raise NotImplementedError("write your pallas kernel here")


# speedup vs baseline: 12.5214x; 12.5214x over previous
import functools

import jax
import jax.numpy as jnp
from jax import lax
from jax.experimental import pallas as pl
from jax.experimental.pallas import tpu as pltpu
from jax.experimental.pallas import tpu_sc as plsc

BATCH = 4096
SEQ = 200
EMBED = 128
NUM_CLASSES = 1000
VOCAB = 100000

NUM_WORKERS = 32
SAMPLES_PER_WORKER = BATCH // NUM_WORKERS
CHUNK = 40
CHUNKS_PER_SAMPLE = SEQ // CHUNK
IDX_ROWS_PER_WORKER = SAMPLES_PER_WORKER * CHUNKS_PER_SAMPLE
LANES = 16
VECS = EMBED // LANES
GROUP = 32


def _sc_pool(x2, table):
    mesh = plsc.VectorSubcoreMesh(core_axis_name="c", subcore_axis_name="s")

    @functools.partial(
        pl.kernel,
        out_type=jax.ShapeDtypeStruct((BATCH, EMBED), jnp.float32),
        mesh=mesh,
        scratch_types=[
            pltpu.VMEM((IDX_ROWS_PER_WORKER, CHUNK), jnp.int32),
            pltpu.VMEM((CHUNKS_PER_SAMPLE, CHUNK, EMBED), jnp.float32),
            pltpu.VMEM((GROUP, EMBED), jnp.float32),
        ]
        + [pltpu.SemaphoreType.DMA] * CHUNKS_PER_SAMPLE,
    )
    def k(x_hbm, table_hbm, out_hbm, idx_v, rows_v, acc_v, *sems):
        wid = lax.axis_index("s") * 2 + lax.axis_index("c")
        idx_base = wid * IDX_ROWS_PER_WORKER

        pltpu.sync_copy(x_hbm.at[pl.ds(idx_base, IDX_ROWS_PER_WORKER)], idx_v)

        def issue(sample, c):
            pltpu.async_copy(
                table_hbm.at[idx_v.at[sample * CHUNKS_PER_SAMPLE + c]],
                rows_v.at[c],
                sems[c],
            )

        def drain(c):
            pltpu.make_async_copy(
                table_hbm.at[pl.ds(0, CHUNK)], rows_v.at[c], sems[c]
            ).wait()

        def accum_chunk(c, acc):
            def body(r, a):
                return tuple(
                    a[j] + rows_v[c, r, pl.ds(LANES * j, LANES)]
                    for j in range(VECS)
                )
            return lax.fori_loop(0, CHUNK, body, acc)

        for c in range(CHUNKS_PER_SAMPLE):
            issue(0, c)

        steps_per_group = GROUP

        def step(s, carry):
            acc = tuple(jnp.zeros((LANES,), jnp.float32) for _ in range(VECS))
            for c in range(CHUNKS_PER_SAMPLE):
                drain(c)
                acc = accum_chunk(c, acc)

                @pl.when(s + 1 < SAMPLES_PER_WORKER)
                def _prefetch():
                    issue(s + 1, c)

            s_mod = lax.rem(s, steps_per_group)
            for j in range(VECS):
                acc_v[s_mod, pl.ds(LANES * j, LANES)] = acc[j] * (1.0 / SEQ)

            @pl.when(s_mod == steps_per_group - 1)
            def _flush():
                g = s // steps_per_group
                pltpu.sync_copy(
                    acc_v,
                    out_hbm.at[pl.ds(wid * SAMPLES_PER_WORKER + g * GROUP, GROUP)],
                )
            return carry

        lax.fori_loop(0, SAMPLES_PER_WORKER, step, 0)

    return k(x2, table)


def _tc_matmul(pooled, W, b2):
    BB = 512

    def body(p_ref, w_ref, b_ref, o_ref):
        o_ref[...] = (
            jnp.dot(p_ref[...], w_ref[...], preferred_element_type=jnp.float32)
            + b_ref[...]
        )

    return pl.pallas_call(
        body,
        grid=(BATCH // BB,),
        in_specs=[
            pl.BlockSpec((BB, EMBED), lambda i: (i, 0)),
            pl.BlockSpec((EMBED, NUM_CLASSES), lambda i: (0, 0)),
            pl.BlockSpec((1, NUM_CLASSES), lambda i: (0, 0)),
        ],
        out_specs=pl.BlockSpec((BB, NUM_CLASSES), lambda i: (i, 0)),
        out_shape=jax.ShapeDtypeStruct((BATCH, NUM_CLASSES), jnp.float32),
    )(pooled, W, b2)


def kernel(x, table, W, b):
    x2 = x.astype(jnp.int32).reshape(BATCH * CHUNKS_PER_SAMPLE, CHUNK)
    pooled = _sc_pool(x2, table)
    return _tc_matmul(pooled, W, b.reshape(1, NUM_CLASSES))


# speedup vs baseline: 12.5303x; 1.0007x over previous
import functools

import jax
import jax.numpy as jnp
from jax import lax
from jax.experimental import pallas as pl
from jax.experimental.pallas import tpu as pltpu
from jax.experimental.pallas import tpu_sc as plsc

BATCH = 4096
SEQ = 200
EMBED = 128
NUM_CLASSES = 1000
VOCAB = 100000

NUM_WORKERS = 32
SAMPLES_PER_WORKER = BATCH // NUM_WORKERS
CHUNK = 40
CHUNKS_PER_SAMPLE = SEQ // CHUNK
IDX_ROWS_PER_WORKER = SAMPLES_PER_WORKER * CHUNKS_PER_SAMPLE
LANES = 16
VECS = EMBED // LANES
GROUP = 32


def _sc_pool(x2, table):
    mesh = plsc.VectorSubcoreMesh(core_axis_name="c", subcore_axis_name="s")

    @functools.partial(
        pl.kernel,
        out_type=jax.ShapeDtypeStruct((BATCH, EMBED), jnp.float32),
        mesh=mesh,
        scratch_types=[
            pltpu.VMEM((IDX_ROWS_PER_WORKER, CHUNK), jnp.int32),
            pltpu.VMEM((CHUNKS_PER_SAMPLE, CHUNK, EMBED), jnp.float32),
            pltpu.VMEM((GROUP, EMBED), jnp.float32),
        ]
        + [pltpu.SemaphoreType.DMA] * CHUNKS_PER_SAMPLE,
    )
    def k(x_hbm, table_hbm, out_hbm, idx_v, rows_v, acc_v, *sems):
        wid = lax.axis_index("s") * 2 + lax.axis_index("c")
        idx_base = wid * IDX_ROWS_PER_WORKER

        pltpu.sync_copy(x_hbm.at[pl.ds(idx_base, IDX_ROWS_PER_WORKER)], idx_v)

        def issue(sample, c):
            pltpu.async_copy(
                table_hbm.at[idx_v.at[sample * CHUNKS_PER_SAMPLE + c]],
                rows_v.at[c],
                sems[c],
            )

        def drain(c):
            pltpu.make_async_copy(
                table_hbm.at[pl.ds(0, CHUNK)], rows_v.at[c], sems[c]
            ).wait()

        def accum_chunk(c, acc):
            @plsc.parallel_loop(0, CHUNK, unroll=8, carry=acc)
            def body(r, a):
                return tuple(
                    a[j] + rows_v[c, r, pl.ds(LANES * j, LANES)]
                    for j in range(VECS)
                )
            return body

        for c in range(CHUNKS_PER_SAMPLE):
            issue(0, c)

        steps_per_group = GROUP

        def step(s, carry):
            acc = tuple(jnp.zeros((LANES,), jnp.float32) for _ in range(VECS))
            for c in range(CHUNKS_PER_SAMPLE):
                drain(c)
                acc = accum_chunk(c, acc)

                @pl.when(s + 1 < SAMPLES_PER_WORKER)
                def _prefetch():
                    issue(s + 1, c)

            s_mod = lax.rem(s, steps_per_group)
            for j in range(VECS):
                acc_v[s_mod, pl.ds(LANES * j, LANES)] = acc[j] * (1.0 / SEQ)

            @pl.when(s_mod == steps_per_group - 1)
            def _flush():
                g = s // steps_per_group
                pltpu.sync_copy(
                    acc_v,
                    out_hbm.at[pl.ds(wid * SAMPLES_PER_WORKER + g * GROUP, GROUP)],
                )
            return carry

        lax.fori_loop(0, SAMPLES_PER_WORKER, step, 0)

    return k(x2, table)


def _tc_matmul(pooled, W, b2):
    BB = 512

    def body(p_ref, w_ref, b_ref, o_ref):
        o_ref[...] = (
            jnp.dot(p_ref[...], w_ref[...], preferred_element_type=jnp.float32)
            + b_ref[...]
        )

    return pl.pallas_call(
        body,
        grid=(BATCH // BB,),
        in_specs=[
            pl.BlockSpec((BB, EMBED), lambda i: (i, 0)),
            pl.BlockSpec((EMBED, NUM_CLASSES), lambda i: (0, 0)),
            pl.BlockSpec((1, NUM_CLASSES), lambda i: (0, 0)),
        ],
        out_specs=pl.BlockSpec((BB, NUM_CLASSES), lambda i: (i, 0)),
        out_shape=jax.ShapeDtypeStruct((BATCH, NUM_CLASSES), jnp.float32),
    )(pooled, W, b2)


def kernel(x, table, W, b):
    x2 = x.astype(jnp.int32).reshape(BATCH * CHUNKS_PER_SAMPLE, CHUNK)
    pooled = _sc_pool(x2, table)
    return _tc_matmul(pooled, W, b.reshape(1, NUM_CLASSES))


# speedup vs baseline: 12.6433x; 1.0090x over previous
import functools

import jax
import jax.numpy as jnp
from jax import lax
from jax.experimental import pallas as pl
from jax.experimental.pallas import tpu as pltpu
from jax.experimental.pallas import tpu_sc as plsc

BATCH = 4096
SEQ = 200
EMBED = 128
NUM_CLASSES = 1000
VOCAB = 100000

NUM_WORKERS = 32
SAMPLES_PER_WORKER = BATCH // NUM_WORKERS
CHUNK = 40
CHUNKS_PER_SAMPLE = SEQ // CHUNK
IDX_ROWS_PER_WORKER = SAMPLES_PER_WORKER * CHUNKS_PER_SAMPLE
LANES = 16
VECS = EMBED // LANES
GROUP = 32


def _sc_pool(x2, table):
    mesh = plsc.VectorSubcoreMesh(core_axis_name="c", subcore_axis_name="s")

    @functools.partial(
        pl.kernel,
        out_type=jax.ShapeDtypeStruct((BATCH, EMBED), jnp.float32),
        mesh=mesh,
        scratch_types=[
            pltpu.VMEM((IDX_ROWS_PER_WORKER, CHUNK), jnp.int32),
            pltpu.VMEM((CHUNKS_PER_SAMPLE, CHUNK, EMBED), jnp.float32),
            pltpu.VMEM((GROUP, EMBED), jnp.float32),
        ]
        + [pltpu.SemaphoreType.DMA] * CHUNKS_PER_SAMPLE,
        compiler_params=pltpu.CompilerParams(use_tc_tiling_on_sc=False),
    )
    def k(x_hbm, table_hbm, out_hbm, idx_v, rows_v, acc_v, *sems):
        wid = lax.axis_index("s") * 2 + lax.axis_index("c")
        idx_base = wid * IDX_ROWS_PER_WORKER

        pltpu.sync_copy(x_hbm.at[pl.ds(idx_base, IDX_ROWS_PER_WORKER)], idx_v)

        def issue(sample, c):
            pltpu.async_copy(
                table_hbm.at[idx_v.at[sample * CHUNKS_PER_SAMPLE + c]],
                rows_v.at[c],
                sems[c],
            )

        def drain(c):
            pltpu.make_async_copy(
                table_hbm.at[pl.ds(0, CHUNK)], rows_v.at[c], sems[c]
            ).wait()

        def accum_chunk(c, acc):
            @plsc.parallel_loop(0, CHUNK, unroll=8, carry=acc)
            def body(r, a):
                out = []
                for g in range(VECS):
                    w = rows_v[c, r, pl.ds(LANES * g, LANES)]
                    out.append(a[g] + w)
                return tuple(out)
            return body

        for c in range(CHUNKS_PER_SAMPLE):
            issue(0, c)

        steps_per_group = GROUP

        def step(s, carry):
            acc = tuple(jnp.zeros((LANES,), jnp.float32) for _ in range(VECS))
            for c in range(CHUNKS_PER_SAMPLE):
                drain(c)
                acc = accum_chunk(c, acc)

                @pl.when(s + 1 < SAMPLES_PER_WORKER)
                def _prefetch():
                    issue(s + 1, c)

            s_mod = lax.rem(s, steps_per_group)
            for j in range(VECS):
                acc_v[s_mod, pl.ds(LANES * j, LANES)] = acc[j] * (1.0 / SEQ)

            @pl.when(s_mod == steps_per_group - 1)
            def _flush():
                g = s // steps_per_group
                pltpu.sync_copy(
                    acc_v,
                    out_hbm.at[pl.ds(wid * SAMPLES_PER_WORKER + g * GROUP, GROUP)],
                )
            return carry

        lax.fori_loop(0, SAMPLES_PER_WORKER, step, 0)

    return k(x2, table)


def _tc_matmul(pooled, W, b2):
    BB = 512

    def body(p_ref, w_ref, b_ref, o_ref):
        o_ref[...] = (
            jnp.dot(p_ref[...], w_ref[...], preferred_element_type=jnp.float32)
            + b_ref[...]
        )

    return pl.pallas_call(
        body,
        grid=(BATCH // BB,),
        in_specs=[
            pl.BlockSpec((BB, EMBED), lambda i: (i, 0)),
            pl.BlockSpec((EMBED, NUM_CLASSES), lambda i: (0, 0)),
            pl.BlockSpec((1, NUM_CLASSES), lambda i: (0, 0)),
        ],
        out_specs=pl.BlockSpec((BB, NUM_CLASSES), lambda i: (i, 0)),
        out_shape=jax.ShapeDtypeStruct((BATCH, NUM_CLASSES), jnp.float32),
    )(pooled, W, b2)


def kernel(x, table, W, b):
    x2 = x.astype(jnp.int32).reshape(BATCH * CHUNKS_PER_SAMPLE, CHUNK)
    pooled = _sc_pool(x2, table)
    return _tc_matmul(pooled, W, b.reshape(1, NUM_CLASSES))
